# baseline (speedup 1.0000x reference)
import functools

import jax
import jax.numpy as jnp
from jax import lax
from jax.experimental import pallas as pl
from jax.experimental.pallas import tpu as pltpu
from jax.experimental.pallas import tpu_sc as plsc

EPS = 1e-5
LANE = 128
SUB = 8


def _sc_gather_fm(tableT, action):
    D, V = tableT.shape
    B = action.shape[0]
    info = plsc.get_sparse_core_info()
    nc, ns, nl = info.num_cores, info.num_subcores, info.num_lanes
    nw = nc * ns
    b_per_w = B // nw
    w_words = b_per_w * D
    tiles_per_row = (V + LANE - 1) // LANE
    trw = tiles_per_row * LANE * SUB

    mesh = plsc.VectorSubcoreMesh(core_axis_name="c", subcore_axis_name="s")

    @functools.partial(
        pl.kernel,
        mesh=mesh,
        compiler_params=pltpu.CompilerParams(
            use_tc_tiling_on_sc=False, disable_bounds_checks=True
        ),
        out_type=jax.ShapeDtypeStruct((D * B,), jnp.float32),
        scratch_types=[
            pltpu.VMEM((b_per_w,), jnp.int32),
            pltpu.VMEM((w_words,), jnp.float32),
            pltpu.SemaphoreType.DMA,
        ],
    )
    def gather_kernel(tableT_hbm, idx_hbm, out_hbm, idx_v, vals_v, sem):
        wid = lax.axis_index("s") * nc + lax.axis_index("c")
        base = wid * b_per_w
        pltpu.sync_copy(idx_hbm.at[pl.ds(base, b_per_w)], idx_v)

        def fire(f, carry):
            pltpu.async_copy(
                tableT_hbm.at[f].at[idx_v],
                vals_v.at[pl.ds(f * b_per_w, b_per_w)],
                sem,
            )
            return carry

        def drain(f, carry):
            pltpu.make_async_copy(
                tableT_hbm.at[f].at[idx_v],
                vals_v.at[pl.ds(f * b_per_w, b_per_w)],
                sem,
            ).wait()
            return carry

        def flush(f, carry):
            pltpu.sync_copy(
                vals_v.at[pl.ds(f * b_per_w, b_per_w)],
                out_hbm.at[pl.ds(f * B + base, b_per_w)],
            )
            return carry

        lax.fori_loop(0, D, fire, 0)
        lax.fori_loop(0, D, drain, 0)
        lax.fori_loop(0, D, flush, 0)

    return gather_kernel(tableT, action)


def _bn_tanh(x, g, b, n_batch):
    d, n = x.shape
    inv_n = 1.0 / n_batch

    def body(x_ref, g_ref, b_ref, o_ref):
        v = x_ref[...]
        mean = jnp.sum(v, axis=1, keepdims=True) * inv_n
        ex2 = jnp.sum(v * v, axis=1, keepdims=True) * inv_n
        var = ex2 - mean * mean
        scale = g_ref[...] * lax.rsqrt(var + EPS)
        o_ref[...] = jnp.tanh((v - mean) * scale + b_ref[...])

    return pl.pallas_call(
        body,
        out_shape=jax.ShapeDtypeStruct((d, n), jnp.float32),
    )(x, g, b)


@jax.jit
def kernel(action, table, gamma, beta):
    V, D = table.shape
    B = action.shape[0]
    flat = _sc_gather_fm(table.T, action)
    x = flat.reshape(D, B)
    out = _bn_tanh(x, gamma.reshape(D, 1), beta.reshape(D, 1), B)
    return out.T


# speedup vs baseline: 4.9012x; 4.9012x over previous
import functools

import jax
import jax.numpy as jnp
from jax import lax
from jax.experimental import pallas as pl
from jax.experimental.pallas import tpu as pltpu
from jax.experimental.pallas import tpu_sc as plsc

EPS = 1e-5


def _sc_gather(table, action):
    V, D = table.shape
    B = action.shape[0]
    info = plsc.get_sparse_core_info()
    nw = info.num_cores * info.num_subcores
    b_per_w = B // nw

    mesh = plsc.VectorSubcoreMesh(core_axis_name="c", subcore_axis_name="s")

    @functools.partial(
        pl.kernel,
        mesh=mesh,
        compiler_params=pltpu.CompilerParams(use_tc_tiling_on_sc=False),
        out_type=jax.ShapeDtypeStruct((B, D), jnp.float32),
        scratch_types=[
            pltpu.VMEM((b_per_w,), jnp.int32),
            pltpu.VMEM((b_per_w, D), jnp.float32),
            pltpu.SemaphoreType.DMA,
        ],
    )
    def gather_kernel(table_hbm, idx_hbm, out_hbm, idx_v, rows_v, sem):
        wid = lax.axis_index("s") * info.num_cores + lax.axis_index("c")
        base = wid * b_per_w
        pltpu.sync_copy(idx_hbm.at[pl.ds(base, b_per_w)], idx_v)
        pltpu.async_copy(table_hbm.at[idx_v], rows_v, sem).wait()
        pltpu.sync_copy(rows_v, out_hbm.at[pl.ds(base, b_per_w)])

    return gather_kernel(table, action)


def _bn_tanh(x, g128, b128, n_rows, n_feat):
    rows, lanes = x.shape
    inv_n = 1.0 / n_rows

    def body(x_ref, g_ref, b_ref, o_ref):
        v = x_ref[...]
        s = jnp.sum(v, axis=0, keepdims=True)
        ss = jnp.sum(v * v, axis=0, keepdims=True)
        li = lax.broadcasted_iota(jnp.int32, (lanes, lanes), 0) % n_feat
        lj = lax.broadcasted_iota(jnp.int32, (lanes, lanes), 1) % n_feat
        fold = (li == lj).astype(jnp.float32)
        mean = jnp.dot(s, fold, preferred_element_type=jnp.float32) * inv_n
        ex2 = jnp.dot(ss, fold, preferred_element_type=jnp.float32) * inv_n
        var = ex2 - mean * mean
        scale = g_ref[...] * lax.rsqrt(var + EPS)
        o_ref[...] = jnp.tanh((v - mean) * scale + b_ref[...])

    return pl.pallas_call(
        body,
        out_shape=jax.ShapeDtypeStruct((rows, lanes), jnp.float32),
    )(x, g128, b128)


@jax.jit
def kernel(action, table, gamma, beta):
    V, D = table.shape
    B = action.shape[0]
    lanes = 128
    groups = lanes // D
    gathered = _sc_gather(table, action)
    x = gathered.reshape(B * D // lanes, lanes)
    g = jnp.tile(gamma, groups).reshape(1, lanes)
    b = jnp.tile(beta, groups).reshape(1, lanes)
    out = _bn_tanh(x, g, b, B, D)
    return out.reshape(B, D)
